# Initial kernel scaffold; baseline (speedup 1.0000x reference)
import jax
import jax.numpy as jnp
from jax.experimental import pallas as pl


def kernel(edge_index, messages, num_nodes, tensors, W1, b1, W2, b2, W3, b3):
    raise NotImplementedError("write your pallas kernel here")


# speedup vs baseline: 1.4144x; 1.4144x over previous
import functools

import jax
import jax.numpy as jnp
from jax import lax
from jax.experimental import pallas as pl
from jax.experimental.pallas import tpu as pltpu
from jax.experimental.pallas import tpu_sc as plsc

NC = 2
NS = 16
NW = NC * NS
L = 16
CH = 128


def _softplus(x):
    return jnp.maximum(x, 0.0) + jnp.log1p(jnp.exp(-jnp.abs(x)))


def _mesh():
    return plsc.VectorSubcoreMesh(
        core_axis_name="c", subcore_axis_name="s", num_cores=NC, num_subcores=NS
    )


def _tc1_edge_projections(messages, W1a, W1b, b1r):
    E, MS = messages.shape
    bs = 1000
    assert E % bs == 0

    def body(msg, w1a, w1b, b1, a_out, b_out):
        m = msg[...]
        a_out[...] = jnp.dot(m, w1a[...], preferred_element_type=jnp.float32)
        b_out[...] = jnp.dot(m, w1b[...], preferred_element_type=jnp.float32) + b1[...]

    return pl.pallas_call(
        body,
        grid=(E // bs,),
        in_specs=[
            pl.BlockSpec((bs, MS), lambda i: (i, 0)),
            pl.BlockSpec((MS, MS), lambda i: (0, 0)),
            pl.BlockSpec((MS, MS), lambda i: (0, 0)),
            pl.BlockSpec((1, MS), lambda i: (0, 0)),
        ],
        out_specs=[
            pl.BlockSpec((bs, MS), lambda i: (i, 0)),
            pl.BlockSpec((bs, MS), lambda i: (i, 0)),
        ],
        out_shape=[
            jax.ShapeDtypeStruct((E, MS), jnp.float32),
            jax.ShapeDtypeStruct((E, MS), jnp.float32),
        ],
    )(messages, W1a, W1b, b1r)


def _sc_gather(A, B, idx_kj, idx_ji, tcap_vec, TMAX):
    E, MS = A.shape

    out_type = jax.ShapeDtypeStruct((TMAX, MS), jnp.float32)
    scratch = [
        pltpu.VMEM((L,), jnp.int32),
        pltpu.VMEM((CH,), jnp.int32),
        pltpu.VMEM((CH,), jnp.int32),
        pltpu.VMEM((CH, MS), jnp.float32),
        pltpu.VMEM((CH, MS), jnp.float32),
        pltpu.SemaphoreType.DMA,
        pltpu.SemaphoreType.DMA,
    ]

    @functools.partial(
        pl.kernel, out_type=out_type, mesh=_mesh(), scratch_types=scratch,
        compiler_params=pltpu.CompilerParams(needs_layout_passes=False))
    def k(A_h, B_h, ikj_h, iji_h, tcap_h, S_h,
          tcv, ikj, iji, bufA, bufB, s0, s1):
        cid = lax.axis_index("c")
        sid = lax.axis_index("s")
        w = sid * NC + cid
        pltpu.sync_copy(tcap_h, tcv)
        tcap = tcv[...][0]
        nchunks = (tcap + (CH - 1)) // CH
        niter = jnp.maximum(nchunks - w + (NW - 1), 0) // NW

        def body(it, _):
            t0 = (w + it * NW) * CH
            pltpu.sync_copy(ikj_h.at[pl.ds(t0, CH)], ikj)
            pltpu.sync_copy(iji_h.at[pl.ds(t0, CH)], iji)
            cA = pltpu.async_copy(A_h.at[ikj], bufA, s0)
            cB = pltpu.async_copy(B_h.at[iji], bufB, s1)
            cA.wait()
            cB.wait()

            def row(r, _):
                for j in range(MS // L):
                    sl = pl.ds(j * L, L)
                    bufA[r, sl] = bufA[r, sl] + bufB[r, sl]
                return 0

            lax.fori_loop(0, CH, row, 0)
            pltpu.sync_copy(bufA, S_h.at[pl.ds(t0, CH)])
            return 0

        lax.fori_loop(0, niter, body, 0)

    return k(A, B, idx_kj, idx_ji, tcap_vec)


def _tc2_mlp(S, W2, b2r, w3r, b3r, tcap1, TMAX):
    MS = S.shape[1]
    bs = 1024
    assert TMAX % bs == 0
    rows = bs // 128

    def body(tc_ref, s_b, w2, b2, w3, b3, out_b):
        i = pl.program_id(0)

        @pl.when(i * bs < tc_ref[0])
        def _():
            h = _softplus(s_b[...])
            h = _softplus(jnp.dot(h, w2[...], preferred_element_type=jnp.float32) + b2[...])
            m = jnp.sum(h * w3[...], axis=1, keepdims=True) + b3[...]
            out_b[...] = m.reshape(rows, 128)

    def blk(i, tc):
        last = jnp.maximum((tc[0] + bs - 1) // bs, 1) - 1
        return (jnp.minimum(i, last), 0)

    grid_spec = pltpu.PrefetchScalarGridSpec(
        num_scalar_prefetch=1,
        grid=(TMAX // bs,),
        in_specs=[
            pl.BlockSpec((bs, MS), blk),
            pl.BlockSpec((MS, MS), lambda i, tc: (0, 0)),
            pl.BlockSpec((1, MS), lambda i, tc: (0, 0)),
            pl.BlockSpec((1, MS), lambda i, tc: (0, 0)),
            pl.BlockSpec((1, 1), lambda i, tc: (0, 0)),
        ],
        out_specs=pl.BlockSpec((rows, 128), blk),
    )
    return pl.pallas_call(
        body,
        grid_spec=grid_spec,
        out_shape=jax.ShapeDtypeStruct((TMAX // 128, 128), jnp.float32),
    )(tcap1, S, W2, b2r, w3r, b3r)


def _sc_scatter(m2d, idx_j, idx_i, idx_k, tens_flat, tcap_vec, zeros_init, N, NPAD):
    out_type = jax.ShapeDtypeStruct((NC, NPAD, 128), jnp.float32)
    rpt = NPAD // NS
    scratch = [
        pltpu.VMEM((L,), jnp.int32),
        pltpu.VMEM((CH,), jnp.int32),
        pltpu.VMEM((CH,), jnp.int32),
        pltpu.VMEM((CH,), jnp.int32),
        pltpu.VMEM((CH,), jnp.float32),
        pltpu.VMEM((CH, 128), jnp.float32),
        pltpu.VMEM((tens_flat.shape[0],), jnp.float32),
        pltpu.VMEM_SHARED((NPAD, 128), jnp.float32),
    ]

    @functools.partial(
        pl.kernel, out_type=out_type, mesh=_mesh(), scratch_types=scratch,
        compiler_params=pltpu.CompilerParams(needs_layout_passes=False))
    def k(m_h, ij_h, ii_h, ik_h, tf_h, tcap_h, z_h, out_h,
          tcv, ijv, iiv, ikv, mv, rows, tfv, acc):
        cid = lax.axis_index("c")
        sid = lax.axis_index("s")
        w = sid * NC + cid
        pltpu.sync_copy(tcap_h, tcv)
        tcap = tcv[...][0]
        pltpu.sync_copy(tf_h, tfv)
        pltpu.sync_copy(z_h.at[pl.ds(sid * rpt, rpt)], acc.at[pl.ds(sid * rpt, rpt)])

        iota = lax.iota(jnp.int32, L)
        rep4 = iota // 4
        til4 = iota - 4 * rep4
        ones = jnp.full((L,), 1.0, jnp.float32)
        zv = jnp.zeros((L,), jnp.float32)
        del iota

        def fill(r, _):
            rows[r, pl.ds(L, L)] = ones
            for q in range(2, 8):
                rows[r, pl.ds(q * L, L)] = zv
            return 0

        lax.fori_loop(0, CH, fill, 0)
        plsc.subcore_barrier()

        nchunks = (tcap + (CH - 1)) // CH
        niter = jnp.maximum(nchunks - w + (NW - 1), 0) // NW

        def body(it, _):
            c = w + it * NW
            t0 = c * CH
            pltpu.sync_copy(ij_h.at[pl.ds(t0, CH)], ijv)
            pltpu.sync_copy(ii_h.at[pl.ds(t0, CH)], iiv)
            pltpu.sync_copy(ik_h.at[pl.ds(t0, CH)], ikv)
            pltpu.sync_copy(m_h.at[c], mv)

            def trip(t, _):
                bc = jnp.broadcast_to(t, (L,))
                ii_s = plsc.load_gather(iiv, [bc])
                ik_s = plsc.load_gather(ikv, [bc])
                ti = plsc.load_gather(tfv, [ii_s * 4 + rep4])
                tk = plsc.load_gather(tfv, [ik_s * 4 + til4])
                m_s = plsc.load_gather(mv, [bc])
                rows[t, pl.ds(0, L)] = m_s * ti * tk
                return 0

            lax.fori_loop(0, CH, trip, 0)
            pltpu.sync_copy(rows, acc.at[ijv], add=True)
            return 0

        lax.fori_loop(0, niter, body, 0)
        plsc.subcore_barrier()
        pltpu.sync_copy(acc.at[pl.ds(sid * rpt, rpt)], out_h.at[cid, pl.ds(sid * rpt, rpt)])

    return k(m2d, idx_j, idx_i, idx_k, tens_flat, tcap_vec, zeros_init)


def _tc3_combine(partials, N, NPAD):
    def body(p, out):
        x = p[...]
        s = x[0] + x[1]
        out[...] = s[:N, :L] / jnp.maximum(s[:N, L:L + 1], 1.0)

    return pl.pallas_call(
        body,
        out_shape=jax.ShapeDtypeStruct((N, L), jnp.float32),
    )(partials)


def kernel(edge_index, messages, num_nodes, tensors, W1, b1, W2, b2, W3, b3):
    E, MS = messages.shape
    N = tensors.shape[0]
    TMAX = 32 * E

    row, col = edge_index[0], edge_index[1]
    order = jnp.argsort(col)
    indeg = jnp.bincount(jnp.minimum(col, N - 1), length=N)
    ptr = jnp.concatenate([jnp.zeros((1,), indeg.dtype), jnp.cumsum(indeg)])
    num_triplets = indeg[row]
    total = jnp.sum(num_triplets)
    idx_ji = jnp.repeat(jnp.arange(E), num_triplets, total_repeat_length=TMAX)
    cum = jnp.concatenate([jnp.zeros((1,), num_triplets.dtype), jnp.cumsum(num_triplets)])[:-1]
    offset = jnp.arange(TMAX) - cum[idx_ji]
    pos = jnp.clip(ptr[row[idx_ji]] + offset, 0, E - 1)
    idx_kj = order[pos].astype(jnp.int32)
    idx_ji = idx_ji.astype(jnp.int32)
    idx_i = col[idx_ji].astype(jnp.int32)
    idx_k = row[idx_kj].astype(jnp.int32)
    idx_j = row[idx_ji].astype(jnp.int32)

    tcap = jnp.minimum(total, TMAX).astype(jnp.int32)
    tcap_vec = jnp.full((L,), 1, jnp.int32) * tcap
    tcap1 = tcap.reshape(1)

    W1a = W1[:MS]
    W1b = W1[MS:]
    A, B = _tc1_edge_projections(messages, W1a, W1b, b1.reshape(1, MS))

    S = _sc_gather(A, B, idx_kj, idx_ji, tcap_vec, TMAX)

    m2d = _tc2_mlp(S, W2, b2.reshape(1, MS), W3.reshape(1, MS),
                   b3.reshape(1, 1), tcap1, TMAX)

    NPAD = ((N + 1 + NS * 8 - 1) // (NS * 8)) * (NS * 8)
    zeros_init = jnp.zeros((NPAD, 128), jnp.float32)
    tens_flat = tensors.reshape(4 * N)
    idx_j_masked = jnp.where(jnp.arange(TMAX) < tcap, idx_j, N).astype(jnp.int32)
    partials = _sc_scatter(m2d, idx_j_masked, idx_i, idx_k, tens_flat, tcap_vec,
                           zeros_init, N, NPAD)

    mean = _tc3_combine(partials, N, NPAD)
    return mean.reshape(N, 2, 2, 2, 2)


# speedup vs baseline: 77.5738x; 54.8445x over previous
import functools

import jax
import jax.numpy as jnp
from jax import lax
from jax.experimental import pallas as pl
from jax.experimental.pallas import tpu as pltpu
from jax.experimental.pallas import tpu_sc as plsc

NC = 2
NS = 16
NW = NC * NS
L = 16
CH = 128


def _softplus(x):
    return jnp.maximum(x, 0.0) + jnp.log1p(jnp.exp(-jnp.abs(x)))


def _mesh():
    return plsc.VectorSubcoreMesh(
        core_axis_name="c", subcore_axis_name="s", num_cores=NC, num_subcores=NS
    )


def _tc1_edge_projections(messages, W1a, W1b, b1r):
    E, MS = messages.shape
    bs = 1000
    assert E % bs == 0

    def body(msg, w1a, w1b, b1, a_out, b_out):
        m = msg[...]
        a_out[...] = jnp.dot(m, w1a[...], preferred_element_type=jnp.float32)
        b_out[...] = jnp.dot(m, w1b[...], preferred_element_type=jnp.float32) + b1[...]

    return pl.pallas_call(
        body,
        grid=(E // bs,),
        in_specs=[
            pl.BlockSpec((bs, MS), lambda i: (i, 0)),
            pl.BlockSpec((MS, MS), lambda i: (0, 0)),
            pl.BlockSpec((MS, MS), lambda i: (0, 0)),
            pl.BlockSpec((1, MS), lambda i: (0, 0)),
        ],
        out_specs=[
            pl.BlockSpec((bs, MS), lambda i: (i, 0)),
            pl.BlockSpec((bs, MS), lambda i: (i, 0)),
        ],
        out_shape=[
            jax.ShapeDtypeStruct((E, MS), jnp.float32),
            jax.ShapeDtypeStruct((E, MS), jnp.float32),
        ],
    )(messages, W1a, W1b, b1r)


def _sc_index_a(cumfull, pmc, order, tcap_vec, E, TMAX):
    out_type = (
        jax.ShapeDtypeStruct((TMAX,), jnp.int32),
        jax.ShapeDtypeStruct((TMAX,), jnp.int32),
        jax.ShapeDtypeStruct((TMAX,), jnp.int32),
    )
    EP = cumfull.shape[0]
    scratch = [
        pltpu.VMEM((EP,), jnp.int32),
        pltpu.VMEM((E,), jnp.int32),
        pltpu.VMEM((E,), jnp.int32),
        pltpu.VMEM((L,), jnp.int32),
        pltpu.VMEM((CH,), jnp.int32),
        pltpu.VMEM((CH,), jnp.int32),
        pltpu.VMEM((CH,), jnp.int32),
    ]

    @functools.partial(
        pl.kernel, out_type=out_type, mesh=_mesh(), scratch_types=scratch,
        compiler_params=pltpu.CompilerParams(needs_layout_passes=False))
    def k(cf_h, pmc_h, ord_h, tcap_h, kj_h, ji_h, pos_h,
          cfv, pmcv, ordv, tcv, kjb, eb, posb):
        cid = lax.axis_index("c")
        sid = lax.axis_index("s")
        w = sid * NC + cid
        pltpu.sync_copy(tcap_h, tcv)
        tcap = tcv[...][0]
        pltpu.sync_copy(cf_h, cfv)
        pltpu.sync_copy(pmc_h, pmcv)
        pltpu.sync_copy(ord_h, ordv)
        iota = lax.iota(jnp.int32, L)
        nchunks = (tcap + (CH - 1)) // CH
        niter = jnp.maximum(nchunks - w + (NW - 1), 0) // NW

        def body(it, _):
            t0 = (w + it * NW) * CH

            def grp(u, _):
                tvec = jnp.broadcast_to(t0 + u * L, (L,)) + iota
                e = jnp.zeros((L,), jnp.int32)
                for sh in (32768, 16384, 8192, 4096, 2048, 1024, 512, 256,
                           128, 64, 32, 16, 8, 4, 2, 1):
                    cand = jnp.minimum(e + sh, E)
                    m = plsc.load_gather(cfv, [cand])
                    e = jnp.where(m <= tvec, cand, e)
                e = jnp.minimum(e, E - 1)
                pos = plsc.load_gather(pmcv, [e]) + tvec
                pos = jnp.minimum(jnp.maximum(pos, 0), E - 1)
                kj = plsc.load_gather(ordv, [pos])
                sl = pl.ds(u * L, L)
                kjb[sl] = kj
                eb[sl] = e
                posb[sl] = pos
                return 0

            lax.fori_loop(0, CH // L, grp, 0)
            pltpu.sync_copy(kjb, kj_h.at[pl.ds(t0, CH)])
            pltpu.sync_copy(eb, ji_h.at[pl.ds(t0, CH)])
            pltpu.sync_copy(posb, pos_h.at[pl.ds(t0, CH)])
            return 0

        lax.fori_loop(0, niter, body, 0)

    return k(cumfull, pmc, order, tcap_vec)


def _sc_index_b(row, col, rowsorted, e_arr, pos_arr, tcap_vec, N, TMAX):
    E = row.shape[0]
    out_type = (
        jax.ShapeDtypeStruct((TMAX,), jnp.int32),
        jax.ShapeDtypeStruct((TMAX,), jnp.int32),
        jax.ShapeDtypeStruct((TMAX,), jnp.int32),
    )
    scratch = [
        pltpu.VMEM((E,), jnp.int32),
        pltpu.VMEM((E,), jnp.int32),
        pltpu.VMEM((E,), jnp.int32),
        pltpu.VMEM((L,), jnp.int32),
        pltpu.VMEM((CH,), jnp.int32),
        pltpu.VMEM((CH,), jnp.int32),
        pltpu.VMEM((CH,), jnp.int32),
        pltpu.VMEM((CH,), jnp.int32),
        pltpu.VMEM((CH,), jnp.int32),
    ]

    @functools.partial(
        pl.kernel, out_type=out_type, mesh=_mesh(), scratch_types=scratch,
        compiler_params=pltpu.CompilerParams(needs_layout_passes=False))
    def k(row_h, col_h, rs_h, e_h, pos_h, tcap_h, ii_h, ik_h, ijm_h,
          rowv, colv, rsv, tcv, eb, posb, iib, ikb, ijmb):
        cid = lax.axis_index("c")
        sid = lax.axis_index("s")
        w = sid * NC + cid
        pltpu.sync_copy(tcap_h, tcv)
        tcapv = tcv[...]
        tcap = tcapv[0]
        pltpu.sync_copy(row_h, rowv)
        pltpu.sync_copy(col_h, colv)
        pltpu.sync_copy(rs_h, rsv)
        iota = lax.iota(jnp.int32, L)
        nchunks = (tcap + (CH - 1)) // CH
        niter = jnp.maximum(nchunks - w + (NW - 1), 0) // NW

        def body(it, _):
            t0 = (w + it * NW) * CH
            pltpu.sync_copy(e_h.at[pl.ds(t0, CH)], eb)
            pltpu.sync_copy(pos_h.at[pl.ds(t0, CH)], posb)

            def grp(u, _):
                sl = pl.ds(u * L, L)
                tvec = jnp.broadcast_to(t0 + u * L, (L,)) + iota
                e = eb[sl]
                pos = posb[sl]
                iib[sl] = plsc.load_gather(colv, [e])
                ikb[sl] = plsc.load_gather(rsv, [pos])
                j = plsc.load_gather(rowv, [e])
                ijmb[sl] = jnp.where(tvec < tcapv, j, N)
                return 0

            lax.fori_loop(0, CH // L, grp, 0)
            pltpu.sync_copy(iib, ii_h.at[pl.ds(t0, CH)])
            pltpu.sync_copy(ikb, ik_h.at[pl.ds(t0, CH)])
            pltpu.sync_copy(ijmb, ijm_h.at[pl.ds(t0, CH)])
            return 0

        lax.fori_loop(0, niter, body, 0)

    return k(row, col, rowsorted, e_arr, pos_arr, tcap_vec)


def _sc_gather(A, B, idx_kj, idx_ji, tcap_vec, TMAX):
    E, MS = A.shape

    out_type = jax.ShapeDtypeStruct((TMAX, MS), jnp.float32)
    scratch = [
        pltpu.VMEM((L,), jnp.int32),
        pltpu.VMEM((CH,), jnp.int32),
        pltpu.VMEM((CH,), jnp.int32),
        pltpu.VMEM((CH, MS), jnp.float32),
        pltpu.VMEM((CH, MS), jnp.float32),
        pltpu.SemaphoreType.DMA,
        pltpu.SemaphoreType.DMA,
    ]

    @functools.partial(
        pl.kernel, out_type=out_type, mesh=_mesh(), scratch_types=scratch,
        compiler_params=pltpu.CompilerParams(needs_layout_passes=False))
    def k(A_h, B_h, ikj_h, iji_h, tcap_h, S_h,
          tcv, ikj, iji, bufA, bufB, s0, s1):
        cid = lax.axis_index("c")
        sid = lax.axis_index("s")
        w = sid * NC + cid
        pltpu.sync_copy(tcap_h, tcv)
        tcap = tcv[...][0]
        nchunks = (tcap + (CH - 1)) // CH
        niter = jnp.maximum(nchunks - w + (NW - 1), 0) // NW

        def body(it, _):
            t0 = (w + it * NW) * CH
            pltpu.sync_copy(ikj_h.at[pl.ds(t0, CH)], ikj)
            pltpu.sync_copy(iji_h.at[pl.ds(t0, CH)], iji)
            cA = pltpu.async_copy(A_h.at[ikj], bufA, s0)
            cB = pltpu.async_copy(B_h.at[iji], bufB, s1)
            cA.wait()
            cB.wait()

            def row(r, _):
                for j in range(MS // L):
                    sl = pl.ds(j * L, L)
                    bufA[r, sl] = bufA[r, sl] + bufB[r, sl]
                return 0

            lax.fori_loop(0, CH, row, 0)
            pltpu.sync_copy(bufA, S_h.at[pl.ds(t0, CH)])
            return 0

        lax.fori_loop(0, niter, body, 0)

    return k(A, B, idx_kj, idx_ji, tcap_vec)


def _tc2_mlp(S, W2, b2r, w3r, b3r, tcap1, TMAX):
    MS = S.shape[1]
    bs = 1024
    assert TMAX % bs == 0
    rows = bs // 128

    def body(tc_ref, s_b, w2, b2, w3, b3, out_b):
        i = pl.program_id(0)

        @pl.when(i * bs < tc_ref[0])
        def _():
            h = _softplus(s_b[...])
            h = _softplus(jnp.dot(h, w2[...], preferred_element_type=jnp.float32) + b2[...])
            m = jnp.sum(h * w3[...], axis=1, keepdims=True) + b3[...]
            out_b[...] = m.reshape(rows, 128)

    def blk(i, tc):
        last = jnp.maximum((tc[0] + bs - 1) // bs, 1) - 1
        return (jnp.minimum(i, last), 0)

    grid_spec = pltpu.PrefetchScalarGridSpec(
        num_scalar_prefetch=1,
        grid=(TMAX // bs,),
        in_specs=[
            pl.BlockSpec((bs, MS), blk),
            pl.BlockSpec((MS, MS), lambda i, tc: (0, 0)),
            pl.BlockSpec((1, MS), lambda i, tc: (0, 0)),
            pl.BlockSpec((1, MS), lambda i, tc: (0, 0)),
            pl.BlockSpec((1, 1), lambda i, tc: (0, 0)),
        ],
        out_specs=pl.BlockSpec((rows, 128), blk),
    )
    return pl.pallas_call(
        body,
        grid_spec=grid_spec,
        out_shape=jax.ShapeDtypeStruct((TMAX // 128, 128), jnp.float32),
    )(tcap1, S, W2, b2r, w3r, b3r)


def _sc_scatter(m2d, idx_j, idx_i, idx_k, tens_flat, tcap_vec, zeros_init, N, NPAD):
    out_type = jax.ShapeDtypeStruct((NC, NPAD, 128), jnp.float32)
    rpt = NPAD // NS
    scratch = [
        pltpu.VMEM((L,), jnp.int32),
        pltpu.VMEM((CH,), jnp.int32),
        pltpu.VMEM((CH,), jnp.int32),
        pltpu.VMEM((CH,), jnp.int32),
        pltpu.VMEM((CH,), jnp.float32),
        pltpu.VMEM((CH, 128), jnp.float32),
        pltpu.VMEM((tens_flat.shape[0],), jnp.float32),
        pltpu.VMEM_SHARED((NPAD, 128), jnp.float32),
    ]

    @functools.partial(
        pl.kernel, out_type=out_type, mesh=_mesh(), scratch_types=scratch,
        compiler_params=pltpu.CompilerParams(needs_layout_passes=False))
    def k(m_h, ij_h, ii_h, ik_h, tf_h, tcap_h, z_h, out_h,
          tcv, ijv, iiv, ikv, mv, rows, tfv, acc):
        cid = lax.axis_index("c")
        sid = lax.axis_index("s")
        w = sid * NC + cid
        pltpu.sync_copy(tcap_h, tcv)
        tcap = tcv[...][0]
        pltpu.sync_copy(tf_h, tfv)
        pltpu.sync_copy(z_h.at[pl.ds(sid * rpt, rpt)], acc.at[pl.ds(sid * rpt, rpt)])

        iota = lax.iota(jnp.int32, L)
        rep4 = iota // 4
        til4 = iota - 4 * rep4
        ones = jnp.full((L,), 1.0, jnp.float32)
        zv = jnp.zeros((L,), jnp.float32)
        del iota

        def fill(r, _):
            rows[r, pl.ds(L, L)] = ones
            for q in range(2, 8):
                rows[r, pl.ds(q * L, L)] = zv
            return 0

        lax.fori_loop(0, CH, fill, 0)
        plsc.subcore_barrier()

        nchunks = (tcap + (CH - 1)) // CH
        niter = jnp.maximum(nchunks - w + (NW - 1), 0) // NW

        def body(it, _):
            c = w + it * NW
            t0 = c * CH
            pltpu.sync_copy(ij_h.at[pl.ds(t0, CH)], ijv)
            pltpu.sync_copy(ii_h.at[pl.ds(t0, CH)], iiv)
            pltpu.sync_copy(ik_h.at[pl.ds(t0, CH)], ikv)
            pltpu.sync_copy(m_h.at[c], mv)

            def trip(t, _):
                bc = jnp.broadcast_to(t, (L,))
                ii_s = plsc.load_gather(iiv, [bc])
                ik_s = plsc.load_gather(ikv, [bc])
                ti = plsc.load_gather(tfv, [ii_s * 4 + rep4])
                tk = plsc.load_gather(tfv, [ik_s * 4 + til4])
                m_s = plsc.load_gather(mv, [bc])
                rows[t, pl.ds(0, L)] = m_s * ti * tk
                return 0

            lax.fori_loop(0, CH, trip, 0)
            pltpu.sync_copy(rows, acc.at[ijv], add=True)
            return 0

        lax.fori_loop(0, niter, body, 0)
        plsc.subcore_barrier()
        pltpu.sync_copy(acc.at[pl.ds(sid * rpt, rpt)], out_h.at[cid, pl.ds(sid * rpt, rpt)])

    return k(m2d, idx_j, idx_i, idx_k, tens_flat, tcap_vec, zeros_init)


def _tc3_combine(partials, N, NPAD):
    def body(p, out):
        x = p[...]
        s = x[0] + x[1]
        out[...] = s[:N, :L] / jnp.maximum(s[:N, L:L + 1], 1.0)

    return pl.pallas_call(
        body,
        out_shape=jax.ShapeDtypeStruct((N, L), jnp.float32),
    )(partials)


def kernel(edge_index, messages, num_nodes, tensors, W1, b1, W2, b2, W3, b3):
    E, MS = messages.shape
    N = tensors.shape[0]
    TMAX = 32 * E

    row, col = edge_index[0].astype(jnp.int32), edge_index[1].astype(jnp.int32)
    order = jnp.argsort(col).astype(jnp.int32)
    indeg = jnp.bincount(jnp.minimum(col, N - 1), length=N)
    ptr = jnp.concatenate([jnp.zeros((1,), indeg.dtype), jnp.cumsum(indeg)])
    num_triplets = indeg[row]
    cumfull = jnp.concatenate(
        [jnp.zeros((1,), num_triplets.dtype), jnp.cumsum(num_triplets)]).astype(jnp.int32)
    total = cumfull[E]
    pmc = (ptr[row].astype(jnp.int32) - cumfull[:E]).astype(jnp.int32)
    rowsorted = row[order].astype(jnp.int32)

    tcap = jnp.minimum(total, TMAX).astype(jnp.int32)
    tcap_vec = jnp.full((L,), 1, jnp.int32) * tcap
    tcap1 = tcap.reshape(1)

    idx_kj, idx_ji, pos = _sc_index_a(cumfull, pmc, order, tcap_vec, E, TMAX)
    idx_i, idx_k, idx_j_masked = _sc_index_b(row, col, rowsorted, idx_ji, pos,
                                             tcap_vec, N, TMAX)

    W1a = W1[:MS]
    W1b = W1[MS:]
    A, B = _tc1_edge_projections(messages, W1a, W1b, b1.reshape(1, MS))

    S = _sc_gather(A, B, idx_kj, idx_ji, tcap_vec, TMAX)

    m2d = _tc2_mlp(S, W2, b2.reshape(1, MS), W3.reshape(1, MS),
                   b3.reshape(1, 1), tcap1, TMAX)

    NPAD = ((N + 1 + NS * 8 - 1) // (NS * 8)) * (NS * 8)
    zeros_init = jnp.zeros((NPAD, 128), jnp.float32)
    tens_flat = tensors.reshape(4 * N)
    partials = _sc_scatter(m2d, idx_j_masked, idx_i, idx_k, tens_flat, tcap_vec,
                           zeros_init, N, NPAD)

    mean = _tc3_combine(partials, N, NPAD)
    return mean.reshape(N, 2, 2, 2, 2)
